# baseline (speedup 1.0000x reference)
import jax
import jax.numpy as jnp
from jax import lax
from jax.experimental import pallas as pl
from jax.experimental.pallas import tpu as pltpu
from jax.experimental.pallas import tpu_sc as plsc

NUM_EMB = 1000000
DIM = 64
TOTAL = 16384 * 20

_info = plsc.get_sparse_core_info()
NC, NS = _info.num_cores, _info.num_subcores
NW = NC * NS
PER_W = TOTAL // NW
CHUNK = 2048
N_CHUNK = PER_W // CHUNK
SUB = 128
N_SUB = CHUNK // SUB


def _sc_gather(idx_hbm, table_hbm, out_hbm, idx_v, rows_v, sem):
  wid = lax.axis_index("s") * NC + lax.axis_index("c")

  def body(c, carry):
    base = wid * PER_W + c * CHUNK
    pltpu.sync_copy(idx_hbm.at[wid, pl.ds(c * N_SUB, N_SUB)], idx_v)
    for j in range(N_SUB):
      pltpu.async_copy(
          table_hbm.at[idx_v.at[j]],
          rows_v.at[pl.ds(j * SUB, SUB)],
          sem,
      )
    for j in range(N_SUB):
      pltpu.make_async_copy(
          table_hbm.at[idx_v.at[j]],
          rows_v.at[pl.ds(j * SUB, SUB)],
          sem,
      ).wait()
    pltpu.sync_copy(rows_v, out_hbm.at[pl.ds(base, CHUNK)])
    return carry

  lax.fori_loop(0, N_CHUNK, body, 0)


@jax.jit
def _gather_rows(idx, table):
  mesh = plsc.VectorSubcoreMesh(core_axis_name="c", subcore_axis_name="s")
  k = pl.kernel(
      _sc_gather,
      mesh=mesh,
      out_type=jax.ShapeDtypeStruct((TOTAL, DIM), jnp.int8),
      scratch_types=[
          pltpu.VMEM((N_SUB, SUB), jnp.int32),
          pltpu.VMEM((CHUNK, DIM), jnp.int8),
          pltpu.SemaphoreType.DMA,
      ],
      compiler_params=pltpu.CompilerParams(use_tc_tiling_on_sc=False),
  )
  return k(idx, table)


def _dequant_body(x_ref, s_ref, o_ref):
  o_ref[...] = (x_ref[...] * s_ref[0:1, :]).astype(jnp.bfloat16)


@jax.jit
def _dequant(rows, scaler):
  n = rows.shape[0]
  x = rows.reshape(n // 2, 2 * DIM)
  s = jnp.broadcast_to(
      jnp.tile(scaler.astype(jnp.float32), 2).reshape(1, 2 * DIM),
      (8, 2 * DIM),
  )
  blk = 2048
  return pl.pallas_call(
      _dequant_body,
      grid=(x.shape[0] // blk,),
      in_specs=[
          pl.BlockSpec((blk, 2 * DIM), lambda i: (i, 0)),
          pl.BlockSpec((8, 2 * DIM), lambda i: (0, 0)),
      ],
      out_specs=pl.BlockSpec((blk, 2 * DIM), lambda i: (i, 0)),
      out_shape=jax.ShapeDtypeStruct((x.shape[0], 2 * DIM), jnp.bfloat16),
  )(x, s)


def kernel(input, weight, weight_scaler):
  b, h = input.shape
  idx = input.reshape(NW, PER_W // SUB, SUB).astype(jnp.int32)
  rows = _gather_rows(idx, weight)
  out = _dequant(rows, weight_scaler)
  return out.reshape(b, h, DIM)


# speedup vs baseline: 1.0003x; 1.0003x over previous
import jax
import jax.numpy as jnp
from jax import lax
from jax.experimental import pallas as pl
from jax.experimental.pallas import tpu as pltpu
from jax.experimental.pallas import tpu_sc as plsc

NUM_EMB = 1000000
DIM = 64
TOTAL = 16384 * 20

_info = plsc.get_sparse_core_info()
NC, NS = _info.num_cores, _info.num_subcores
NW = NC * NS
PER_W = TOTAL // NW
CHUNK = 2048
N_CHUNK = PER_W // CHUNK
SUB = 128
N_SUB = CHUNK // SUB


def _sc_gather(idx_hbm, table_hbm, out_hbm, idx_v, rows_v, sem):
  wid = lax.axis_index("s") * NC + lax.axis_index("c")

  def body(c, carry):
    base = wid * PER_W + c * CHUNK
    pltpu.sync_copy(idx_hbm.at[wid, pl.ds(c * N_SUB, N_SUB)], idx_v)
    for j in range(N_SUB):
      pltpu.async_copy(
          table_hbm.at[idx_v.at[j]],
          rows_v.at[pl.ds(j * SUB, SUB)],
          sem,
      )
    for j in range(N_SUB):
      pltpu.make_async_copy(
          table_hbm.at[idx_v.at[j]],
          rows_v.at[pl.ds(j * SUB, SUB)],
          sem,
      ).wait()
    pltpu.sync_copy(rows_v, out_hbm.at[pl.ds(base, CHUNK)])
    return carry

  lax.fori_loop(0, N_CHUNK, body, 0)


def _gather_rows(idx, table):
  mesh = plsc.VectorSubcoreMesh(core_axis_name="c", subcore_axis_name="s")
  k = pl.kernel(
      _sc_gather,
      mesh=mesh,
      out_type=jax.ShapeDtypeStruct((TOTAL, DIM), jnp.int8),
      scratch_types=[
          pltpu.VMEM((N_SUB, SUB), jnp.int32),
          pltpu.VMEM((CHUNK, DIM), jnp.int8),
          pltpu.SemaphoreType.DMA,
      ],
      compiler_params=pltpu.CompilerParams(use_tc_tiling_on_sc=False),
  )
  return k(idx, table)


def _dequant_body(x_ref, s_ref, o_ref):
  s = jnp.reshape(s_ref[0:1, 0:DIM], (1, 1, DIM))
  o_ref[...] = (x_ref[...] * s).astype(jnp.bfloat16)


def _dequant(rows, scaler, b, h):
  x = rows.reshape(b, h, DIM)
  s = jnp.broadcast_to(
      jnp.pad(scaler.astype(jnp.float32), (0, 2 * DIM - DIM)).reshape(1, -1),
      (8, 2 * DIM),
  )
  blk = 1024
  return pl.pallas_call(
      _dequant_body,
      grid=(b // blk,),
      in_specs=[
          pl.BlockSpec((blk, h, DIM), lambda i: (i, 0, 0)),
          pl.BlockSpec((8, 2 * DIM), lambda i: (0, 0)),
      ],
      out_specs=pl.BlockSpec((blk, h, DIM), lambda i: (i, 0, 0)),
      out_shape=jax.ShapeDtypeStruct((b, h, DIM), jnp.bfloat16),
  )(x, s)


def kernel(input, weight, weight_scaler):
  b, h = input.shape
  idx = input.reshape(NW, PER_W // SUB, SUB).astype(jnp.int32)
  rows = _gather_rows(idx, weight)
  return _dequant(rows, weight_scaler, b, h)
